# baseline (speedup 1.0000x reference)
import functools

import jax
import jax.numpy as jnp
import numpy as np
from jax import lax
from jax.experimental import pallas as pl
from jax.experimental.pallas import tpu as pltpu
from jax.experimental.pallas import tpu_sc as plsc

VOCAB = 100000
D_MODEL = 768
B = 4
S = 2048

_NC = 2
_NS = 16
_NW = _NC * _NS

_SPW = S // _NW
_LANES = 16
_VPR = D_MODEL // _LANES


def _sinusoidal_pe(max_len, d_model):
    pos = np.arange(max_len, dtype=np.float64)[:, None]
    div = np.exp(
        np.arange(0, d_model, 2, dtype=np.float64) * -(np.log(10000.0) / d_model)
    )
    pe = np.zeros((max_len, d_model), dtype=np.float64)
    pe[:, 0::2] = np.sin(pos * div)
    pe[:, 1::2] = np.cos(pos * div)
    return pe.astype(np.float32)


_PE = _sinusoidal_pe(S, D_MODEL)


_HALF = _SPW // 2
_NCHUNK = 2 * B
_CBYTES = _HALF * D_MODEL * 4


def _sc_body(table_hbm, idx_hbm, pe_hbm, out_hbm,
             idx_v, pe_v, rows2, gsem, osem):
    wid = lax.axis_index("s") * _NC + lax.axis_index("c")
    s0 = wid * _SPW

    for b in range(B):
        pltpu.sync_copy(idx_hbm.at[b, pl.ds(s0, _SPW)],
                        idx_v.at[pl.ds(b * _SPW, _SPW)])

    def issue_gather(k):
        p = jnp.bitwise_and(k, 1)
        pltpu.async_copy(
            table_hbm.at[idx_v.at[pl.ds(k * _HALF, _HALF)]],
            rows2.at[p], gsem)

    issue_gather(0)
    pltpu.sync_copy(pe_hbm.at[pl.ds(s0, _SPW)], pe_v)

    def chunk(k, _):
        p = jnp.bitwise_and(k, 1)

        @pl.when(k >= 1)
        def _():
            pltpu.make_async_copy(
                rows2.at[1 - p], out_hbm.at[0, pl.ds(s0, _HALF)], osem
            ).wait()

        @pl.when(k < _NCHUNK - 1)
        def _():
            issue_gather(k + 1)

        pltpu.make_async_copy(
            table_hbm.at[idx_v.at[pl.ds(k * _HALF, _HALF)]],
            rows2.at[p], gsem).wait()

        def row_add(r, _):
            for j in range(_VPR):
                plsc.addupdate(
                    rows2.at[p, r, pl.ds(j * _LANES, _LANES)],
                    pe_v[p * _HALF + r, pl.ds(j * _LANES, _LANES)],
                )
            return ()

        lax.fori_loop(0, _HALF, row_add, (), unroll=False)

        b = lax.shift_right_logical(k, 1)
        pltpu.async_copy(
            rows2.at[p], out_hbm.at[b, pl.ds(s0 + p * _HALF, _HALF)], osem)
        return ()

    lax.fori_loop(0, _NCHUNK, chunk, (), unroll=False)
    pltpu.make_async_copy(
        rows2.at[1], out_hbm.at[0, pl.ds(s0, _HALF)], osem).wait()


@jax.jit
def _embed(idx, table, pe):
    mesh = plsc.VectorSubcoreMesh(core_axis_name="c", subcore_axis_name="s")
    out = pl.kernel(
        _sc_body,
        out_type=jax.ShapeDtypeStruct((B, S, D_MODEL), jnp.float32),
        mesh=mesh,
        scratch_types=[
            pltpu.VMEM((B * _SPW,), jnp.int32),
            pltpu.VMEM((_SPW, D_MODEL), jnp.float32),
            pltpu.VMEM((2, _HALF, D_MODEL), jnp.float32),
            pltpu.SemaphoreType.DMA,
            pltpu.SemaphoreType.DMA,
        ],
    )(table, idx, pe)
    return out


def kernel(x, table):
    return _embed(x.astype(jnp.int32), table, jnp.asarray(_PE))


# speedup vs baseline: 1.5492x; 1.5492x over previous
import functools

import jax
import jax.numpy as jnp
import numpy as np
from jax import lax
from jax.experimental import pallas as pl
from jax.experimental.pallas import tpu as pltpu
from jax.experimental.pallas import tpu_sc as plsc

VOCAB = 100000
D_MODEL = 768
B = 4
S = 2048

_NC = 2
_NS = 16
_NW = _NC * _NS

_SPW = S // _NW
_LANES = 16
_VPR = D_MODEL // _LANES


def _sinusoidal_pe(max_len, d_model):
    pos = np.arange(max_len, dtype=np.float64)[:, None]
    div = np.exp(
        np.arange(0, d_model, 2, dtype=np.float64) * -(np.log(10000.0) / d_model)
    )
    pe = np.zeros((max_len, d_model), dtype=np.float64)
    pe[:, 0::2] = np.sin(pos * div)
    pe[:, 1::2] = np.cos(pos * div)
    return pe.astype(np.float32)


_PE = _sinusoidal_pe(S, D_MODEL)


_HALF = _SPW // 2
_NCHUNK = 2 * B
_CBYTES = _HALF * D_MODEL * 4


def _sc_body(table_hbm, idx_hbm, pe_hbm, out_hbm,
             idx_v, pe_v, rows2, gsem, osem):
    wid = lax.axis_index("s") * _NC + lax.axis_index("c")
    s0 = wid * _SPW

    for b in range(B):
        pltpu.sync_copy(idx_hbm.at[b, pl.ds(s0, _SPW)],
                        idx_v.at[pl.ds(b * _SPW, _SPW)])

    def issue_gather(k):
        p = jnp.bitwise_and(k, 1)
        pltpu.async_copy(
            table_hbm.at[idx_v.at[pl.ds(k * _HALF, _HALF)]],
            rows2.at[p], gsem)

    issue_gather(0)
    pltpu.sync_copy(pe_hbm.at[pl.ds(s0, _SPW)], pe_v)

    def chunk(k, _):
        p = jnp.bitwise_and(k, 1)

        @pl.when(k >= 1)
        def _():
            pltpu.make_async_copy(
                rows2.at[1 - p], out_hbm.at[0, pl.ds(s0, _HALF)], osem
            ).wait()

        @pl.when(k < _NCHUNK - 1)
        def _():
            issue_gather(k + 1)

        pltpu.make_async_copy(
            table_hbm.at[idx_v.at[pl.ds(k * _HALF, _HALF)]],
            rows2.at[p], gsem).wait()

        def row_add(r, _):
            for g in range(0, _VPR, 8):
                vals = [
                    pe_v[p * _HALF + r, pl.ds((g + j) * _LANES, _LANES)]
                    for j in range(8)
                ]
                for j in range(8):
                    plsc.addupdate(
                        rows2.at[p, r, pl.ds((g + j) * _LANES, _LANES)],
                        vals[j],
                    )
            return ()

        lax.fori_loop(0, _HALF, row_add, (), unroll=False)

        b = lax.shift_right_logical(k, 1)
        pltpu.async_copy(
            rows2.at[p], out_hbm.at[b, pl.ds(s0 + p * _HALF, _HALF)], osem)
        return ()

    lax.fori_loop(0, _NCHUNK, chunk, (), unroll=False)
    pltpu.make_async_copy(
        rows2.at[1], out_hbm.at[0, pl.ds(s0, _HALF)], osem).wait()


@jax.jit
def _embed(idx, table, pe):
    mesh = plsc.VectorSubcoreMesh(core_axis_name="c", subcore_axis_name="s")
    out = pl.kernel(
        _sc_body,
        out_type=jax.ShapeDtypeStruct((B, S, D_MODEL), jnp.float32),
        mesh=mesh,
        scratch_types=[
            pltpu.VMEM((B * _SPW,), jnp.int32),
            pltpu.VMEM((_SPW, D_MODEL), jnp.float32),
            pltpu.VMEM((2, _HALF, D_MODEL), jnp.float32),
            pltpu.SemaphoreType.DMA,
            pltpu.SemaphoreType.DMA,
        ],
    )(table, idx, pe)
    return out


def kernel(x, table):
    return _embed(x.astype(jnp.int32), table, jnp.asarray(_PE))


# speedup vs baseline: 1.5521x; 1.0019x over previous
import functools

import jax
import jax.numpy as jnp
import numpy as np
from jax import lax
from jax.experimental import pallas as pl
from jax.experimental.pallas import tpu as pltpu
from jax.experimental.pallas import tpu_sc as plsc

VOCAB = 100000
D_MODEL = 768
B = 4
S = 2048

_NC = 2
_NS = 16
_NW = _NC * _NS

_SPW = S // _NW
_LANES = 16
_VPR = D_MODEL // _LANES


def _sinusoidal_pe(max_len, d_model):
    pos = np.arange(max_len, dtype=np.float64)[:, None]
    div = np.exp(
        np.arange(0, d_model, 2, dtype=np.float64) * -(np.log(10000.0) / d_model)
    )
    pe = np.zeros((max_len, d_model), dtype=np.float64)
    pe[:, 0::2] = np.sin(pos * div)
    pe[:, 1::2] = np.cos(pos * div)
    return pe.astype(np.float32)


_PE = _sinusoidal_pe(S, D_MODEL)


_HALF = _SPW // 2
_NCHUNK = 2 * B
_CBYTES = _HALF * D_MODEL * 4


def _sc_body(table_hbm, idx_hbm, pe_hbm, out_hbm,
             idx_v, pe_v, rows2, gsem, osem):
    wid = lax.axis_index("s") * _NC + lax.axis_index("c")
    s0 = wid * _SPW

    for b in range(B):
        pltpu.sync_copy(idx_hbm.at[b, pl.ds(s0, _SPW)],
                        idx_v.at[pl.ds(b * _SPW, _SPW)])

    def issue_gather(k):
        p = jnp.bitwise_and(k, 1)
        pltpu.async_copy(
            table_hbm.at[idx_v.at[pl.ds(k * _HALF, _HALF)]],
            rows2.at[p], gsem)

    issue_gather(0)
    pltpu.sync_copy(pe_hbm.at[pl.ds(s0, _SPW)], pe_v)

    def chunk(k, _):
        p = jnp.bitwise_and(k, 1)

        @pl.when(k >= 1)
        def _():
            pltpu.make_async_copy(
                rows2.at[1 - p], out_hbm.at[0, pl.ds(s0, _HALF)], osem
            ).wait()

        @pl.when(k < _NCHUNK - 1)
        def _():
            issue_gather(k + 1)

        pltpu.make_async_copy(
            table_hbm.at[idx_v.at[pl.ds(k * _HALF, _HALF)]],
            rows2.at[p], gsem).wait()

        def row_add(r, _):
            for g in range(0, _VPR, 8):
                vals = [
                    pe_v[p * _HALF + r, pl.ds((g + j) * _LANES, _LANES)]
                    for j in range(8)
                ]
                for j in range(8):
                    plsc.addupdate(
                        rows2.at[p, r, pl.ds((g + j) * _LANES, _LANES)],
                        vals[j],
                    )
            return ()

        lax.fori_loop(0, _HALF, row_add, (), unroll=False)

        b = lax.shift_right_logical(k, 1)
        pltpu.async_copy(
            rows2.at[p], out_hbm.at[b, pl.ds(s0 + p * _HALF, _HALF)], osem)
        return ()

    lax.fori_loop(0, _NCHUNK, chunk, (), unroll=False)
    pltpu.make_async_copy(
        rows2.at[1], out_hbm.at[0, pl.ds(s0, _HALF)], osem).wait()


@jax.jit
def _embed(x, table):
    idx = x.astype(jnp.int32)
    pe = jnp.asarray(_PE)
    mesh = plsc.VectorSubcoreMesh(core_axis_name="c", subcore_axis_name="s")
    out = pl.kernel(
        _sc_body,
        out_type=jax.ShapeDtypeStruct((B, S, D_MODEL), jnp.float32),
        mesh=mesh,
        scratch_types=[
            pltpu.VMEM((B * _SPW,), jnp.int32),
            pltpu.VMEM((_SPW, D_MODEL), jnp.float32),
            pltpu.VMEM((2, _HALF, D_MODEL), jnp.float32),
            pltpu.SemaphoreType.DMA,
            pltpu.SemaphoreType.DMA,
        ],
    )(table, idx, pe)
    return out


def kernel(x, table):
    return _embed(x, table)


# speedup vs baseline: 1.5915x; 1.0254x over previous
import functools

import jax
import jax.numpy as jnp
import numpy as np
from jax import lax
from jax.experimental import pallas as pl
from jax.experimental.pallas import tpu as pltpu
from jax.experimental.pallas import tpu_sc as plsc

VOCAB = 100000
D_MODEL = 768
B = 4
S = 2048

_NC = 2
_NS = 16
_NW = _NC * _NS

_SPW = S // _NW
_LANES = 16
_VPR = D_MODEL // _LANES


def _sinusoidal_pe(max_len, d_model):
    pos = np.arange(max_len, dtype=np.float64)[:, None]
    div = np.exp(
        np.arange(0, d_model, 2, dtype=np.float64) * -(np.log(10000.0) / d_model)
    )
    pe = np.zeros((max_len, d_model), dtype=np.float64)
    pe[:, 0::2] = np.sin(pos * div)
    pe[:, 1::2] = np.cos(pos * div)
    return pe.astype(np.float32)


_PE = _sinusoidal_pe(S, D_MODEL)


_HALF = _SPW // 2
_NCHUNK = 2 * B
_CBYTES = _HALF * D_MODEL * 4


def _sc_body(table_hbm, idx_hbm, pe_hbm, out_hbm,
             idx_v, pe_v, rows2, gsem, osem):
    wid = lax.axis_index("s") * _NC + lax.axis_index("c")
    s0 = wid * _SPW

    for b in range(B):
        pltpu.sync_copy(idx_hbm.at[b, pl.ds(s0, _SPW)],
                        idx_v.at[pl.ds(b * _SPW, _SPW)])

    def issue_gather(k):
        p = jnp.bitwise_and(k, 1)
        pltpu.async_copy(
            table_hbm.at[idx_v.at[pl.ds(k * _HALF, _HALF)]],
            rows2.at[p], gsem)

    issue_gather(0)
    pltpu.sync_copy(pe_hbm.at[pl.ds(s0 * D_MODEL, _SPW * D_MODEL)], pe_v)

    def chunk(k, _):
        p = jnp.bitwise_and(k, 1)

        @pl.when(k >= 1)
        def _():
            pltpu.make_async_copy(
                rows2.at[1 - p], out_hbm.at[0, pl.ds(s0, _HALF)], osem
            ).wait()

        @pl.when(k < _NCHUNK - 1)
        def _():
            issue_gather(k + 1)

        pltpu.make_async_copy(
            table_hbm.at[idx_v.at[pl.ds(k * _HALF, _HALF)]],
            rows2.at[p], gsem).wait()

        def row_add(r, _):
            pbase = (p * _HALF + r) * D_MODEL
            for g in range(0, _VPR, 8):
                vals = [
                    pe_v[pl.ds(pbase + (g + j) * _LANES, _LANES)]
                    for j in range(8)
                ]
                for j in range(8):
                    plsc.addupdate(
                        rows2.at[p, r, pl.ds((g + j) * _LANES, _LANES)],
                        vals[j],
                    )
            return ()

        lax.fori_loop(0, _HALF, row_add, (), unroll=False)

        b = lax.shift_right_logical(k, 1)
        pltpu.async_copy(
            rows2.at[p], out_hbm.at[b, pl.ds(s0 + p * _HALF, _HALF)], osem)
        return ()

    lax.fori_loop(0, _NCHUNK, chunk, (), unroll=False)
    pltpu.make_async_copy(
        rows2.at[1], out_hbm.at[0, pl.ds(s0, _HALF)], osem).wait()


@jax.jit
def _embed(x, table):
    idx = x.astype(jnp.int32)
    pe = jnp.asarray(_PE.reshape(-1))
    mesh = plsc.VectorSubcoreMesh(core_axis_name="c", subcore_axis_name="s")
    out = pl.kernel(
        _sc_body,
        out_type=jax.ShapeDtypeStruct((B, S, D_MODEL), jnp.float32),
        mesh=mesh,
        scratch_types=[
            pltpu.VMEM((B * _SPW,), jnp.int32),
            pltpu.VMEM((_SPW * D_MODEL,), jnp.float32),
            pltpu.VMEM((2, _HALF, D_MODEL), jnp.float32),
            pltpu.SemaphoreType.DMA,
            pltpu.SemaphoreType.DMA,
        ],
    )(table, idx, pe)
    return out


def kernel(x, table):
    return _embed(x, table)
